# Initial kernel scaffold; baseline (speedup 1.0000x reference)
import jax
import jax.numpy as jnp
from jax.experimental import pallas as pl


def kernel(x_user, x_item, ei_ui, ei_iu, W_i_user, b_i_user, W_i_item, b_i_item, Wl_i_ui, bl_i_ui, Wr_i_ui, Wl_i_iu, bl_i_iu, Wr_i_iu, W_f_user, b_f_user, W_f_item, b_f_item, Wl_f_ui, bl_f_ui, Wr_f_ui, Wl_f_iu, bl_f_iu, Wr_f_iu, W_o_user, b_o_user, W_o_item, b_o_item, Wl_o_ui, bl_o_ui, Wr_o_ui, Wl_o_iu, bl_o_iu, Wr_o_iu, W_c_user, b_c_user, W_c_item, b_c_item, Wl_c_ui, bl_c_ui, Wr_c_ui, Wl_c_iu, bl_c_iu, Wr_c_iu):
    raise NotImplementedError("write your pallas kernel here")


# speedup vs baseline: 6.6412x; 6.6412x over previous
import functools

import jax
import jax.numpy as jnp
from jax import lax
from jax.experimental import pallas as pl
from jax.experimental.pallas import tpu as pltpu
from jax.experimental.pallas import tpu_sc as plsc

N = 10000
D = 128
E = 320000
DH = 64
W_ROW = 80
CHUNK = 128
TILES = 16
CHUNKS_PER_TILE = 160
E_PAD = TILES * CHUNKS_PER_TILE * CHUNK
N_PAD = N + 112
ROWS_PER_TILE = N_PAD // TILES
NG = 3
DW = NG * D


def _sc_segment_sums(src_ui, dst_ui, src_iu, dst_iu,
                     xu_a, xu_b, xi_a, xi_b, zrows):
    mesh = plsc.VectorSubcoreMesh(core_axis_name="c", subcore_axis_name="s")
    acc_sds = jax.ShapeDtypeStruct((N_PAD, W_ROW), jnp.float32)

    @functools.partial(
        pl.kernel,
        mesh=mesh,
        out_type=[acc_sds, acc_sds, acc_sds, acc_sds],
        scratch_types=[
            pltpu.VMEM((CHUNKS_PER_TILE, CHUNK), jnp.int32),
            pltpu.VMEM((CHUNKS_PER_TILE, CHUNK), jnp.int32),
            pltpu.VMEM((CHUNK, W_ROW), jnp.float32),
            pltpu.VMEM_SHARED((N_PAD, W_ROW), jnp.float32),
            pltpu.SemaphoreType.DMA,
        ],
        compiler_params=pltpu.CompilerParams(use_tc_tiling_on_sc=False),
    )
    def k(src_ui_h, dst_ui_h, src_iu_h, dst_iu_h,
          xu_a_h, xu_b_h, xi_a_h, xi_b_h, z_h,
          out_ui_a_h, out_ui_b_h, out_iu_a_h, out_iu_b_h,
          src_v, dst_v, rows_v, acc_s, sem):
        core = lax.axis_index("c")
        sid = lax.axis_index("s")
        row_sl = pl.ds(sid * ROWS_PER_TILE, ROWS_PER_TILE)

        pltpu.sync_copy(z_h, acc_s.at[row_sl])

        def run(src_h, dst_h, tables, outs):
            idx_sl = pl.ds(sid * CHUNKS_PER_TILE, CHUNKS_PER_TILE)
            pltpu.sync_copy(src_h.at[idx_sl], src_v)
            pltpu.sync_copy(dst_h.at[idx_sl], dst_v)
            plsc.subcore_barrier()
            for phase, (table_h, out_h) in enumerate(zip(tables, outs)):
                def body(j, carry):
                    pltpu.async_copy(table_h.at[src_v.at[j]], rows_v,
                                     sem).wait()
                    pltpu.sync_copy(rows_v, acc_s.at[dst_v.at[j]], add=True)
                    return carry

                lax.fori_loop(0, CHUNKS_PER_TILE, body, 0)
                plsc.subcore_barrier()
                pltpu.sync_copy(acc_s.at[row_sl], out_h.at[row_sl])
                if phase == 0:
                    pltpu.sync_copy(z_h, acc_s.at[row_sl])
                    plsc.subcore_barrier()

        @pl.when(core == 0)
        def _():
            run(src_ui_h, dst_ui_h, (xu_a_h, xu_b_h),
                (out_ui_a_h, out_ui_b_h))

        @pl.when(core == 1)
        def _():
            run(src_iu_h, dst_iu_h, (xi_a_h, xi_b_h),
                (out_iu_a_h, out_iu_b_h))

    return k(src_ui, dst_ui, src_iu, dst_iu, xu_a, xu_b, xi_a, xi_b, zrows)


def _tc_body(aui_a_ref, aui_b_ref, aiu_a_ref, aiu_b_ref, xu_ref, xi_ref,
             wu_ref, wi_ref, wlui_ref, wrui_ref, wliu_ref, wriu_ref,
             bu_ref, bi_ref,
             hu_ref, hi_ref, cu_ref, ci_ref,
             Au, Bu, Ai, Bi):
    f32 = jnp.float32

    @pl.when(pl.program_id(0) == 0)
    def _():
        for g in range(NG):
            sl = slice(g * D, (g + 1) * D)
            Au[:, sl] = jnp.dot(wi_ref[:, sl], wliu_ref[:, sl],
                                preferred_element_type=f32)
            Bu[:, sl] = jnp.dot(wu_ref[:, sl], wriu_ref[:, sl],
                                preferred_element_type=f32)
            Ai[:, sl] = jnp.dot(wu_ref[:, sl], wlui_ref[:, sl],
                                preferred_element_type=f32)
            Bi[:, sl] = jnp.dot(wi_ref[:, sl], wrui_ref[:, sl],
                                preferred_element_type=f32)

    def halve(a_ref, b_ref, x_ref, A, B, bias_ref):
        inv = 1.0 / jnp.maximum(a_ref[:, DH:DH + 1], 1.0)
        agg_a = a_ref[:, 0:DH] * inv
        agg_b = b_ref[:, 0:DH] * inv
        P = (jnp.dot(agg_a, A[0:DH, :], preferred_element_type=f32)
             + jnp.dot(agg_b, A[DH:D, :], preferred_element_type=f32)
             + jnp.dot(x_ref[:], B[:], preferred_element_type=f32)
             + bias_ref[:])
        i_g = jax.nn.sigmoid(P[:, 0:D])
        t_g = jnp.tanh(P[:, D:2 * D])
        o_g = jax.nn.sigmoid(P[:, 2 * D:3 * D])
        c = i_g * t_g
        h = o_g * jnp.tanh(c)
        return h, c

    h_u, c_u = halve(aiu_a_ref, aiu_b_ref, xu_ref, Au, Bu, bu_ref)
    h_i, c_i = halve(aui_a_ref, aui_b_ref, xi_ref, Ai, Bi, bi_ref)
    hu_ref[:] = h_u
    cu_ref[:] = c_u
    hi_ref[:] = h_i
    ci_ref[:] = c_i


def _tc_dense(aui_a, aui_b, aiu_a, aiu_b, x_user, x_item,
              Wu, Wi, Wl_ui, Wr_ui, Wl_iu, Wr_iu, bias_u, bias_i):
    blk = 2000
    grid = (N // blk,)
    row_spec = pl.BlockSpec((blk, D), lambda k: (k, 0))
    acc_spec = pl.BlockSpec((blk, W_ROW), lambda k: (k, 0))
    w_spec = pl.BlockSpec((D, DW), lambda k: (0, 0))
    b_spec = pl.BlockSpec((1, DW), lambda k: (0, 0))
    out_sds = jax.ShapeDtypeStruct((N, D), jnp.float32)
    return pl.pallas_call(
        _tc_body,
        grid=grid,
        in_specs=[acc_spec, acc_spec, acc_spec, acc_spec, row_spec, row_spec,
                  w_spec, w_spec, w_spec, w_spec, w_spec, w_spec,
                  b_spec, b_spec],
        out_specs=[row_spec, row_spec, row_spec, row_spec],
        out_shape=[out_sds, out_sds, out_sds, out_sds],
        scratch_shapes=[pltpu.VMEM((D, DW), jnp.float32)] * 4,
    )(aui_a, aui_b, aiu_a, aiu_b, x_user, x_item,
      Wu, Wi, Wl_ui, Wr_ui, Wl_iu, Wr_iu, bias_u, bias_i)


def kernel(x_user, x_item, ei_ui, ei_iu, W_i_user, b_i_user, W_i_item, b_i_item, Wl_i_ui, bl_i_ui, Wr_i_ui, Wl_i_iu, bl_i_iu, Wr_i_iu, W_f_user, b_f_user, W_f_item, b_f_item, Wl_f_ui, bl_f_ui, Wr_f_ui, Wl_f_iu, bl_f_iu, Wr_f_iu, W_o_user, b_o_user, W_o_item, b_o_item, Wl_o_ui, bl_o_ui, Wr_o_ui, Wl_o_iu, bl_o_iu, Wr_o_iu, W_c_user, b_c_user, W_c_item, b_c_item, Wl_c_ui, bl_c_ui, Wr_c_ui, Wl_c_iu, bl_c_iu, Wr_c_iu):
    f32 = jnp.float32

    ones_col = jnp.ones((N, 1), f32)
    pad_cols = jnp.zeros((N, W_ROW - DH - 1), f32)
    xu_a = jnp.concatenate([x_user[:, :DH], ones_col, pad_cols], axis=1)
    xu_b = jnp.concatenate([x_user[:, DH:], ones_col, pad_cols], axis=1)
    xi_a = jnp.concatenate([x_item[:, :DH], ones_col, pad_cols], axis=1)
    xi_b = jnp.concatenate([x_item[:, DH:], ones_col, pad_cols], axis=1)

    pad_e = E_PAD - E
    src_pad = jnp.zeros((pad_e,), jnp.int32)
    dst_pad = jnp.full((pad_e,), N, jnp.int32)
    rows2d = (TILES * CHUNKS_PER_TILE, CHUNK)
    src_ui = jnp.concatenate([ei_ui[0], src_pad]).reshape(rows2d)
    dst_ui = jnp.concatenate([ei_ui[1], dst_pad]).reshape(rows2d)
    src_iu = jnp.concatenate([ei_iu[0], src_pad]).reshape(rows2d)
    dst_iu = jnp.concatenate([ei_iu[1], dst_pad]).reshape(rows2d)

    zrows = jnp.zeros((ROWS_PER_TILE, W_ROW), f32)

    aui_a, aui_b, aiu_a, aiu_b = _sc_segment_sums(
        src_ui, dst_ui, src_iu, dst_iu, xu_a, xu_b, xi_a, xi_b, zrows)

    Wu = jnp.concatenate([W_i_user, W_c_user, W_o_user], axis=1)
    Wi = jnp.concatenate([W_i_item, W_c_item, W_o_item], axis=1)
    Wl_ui_s = jnp.concatenate([Wl_i_ui, Wl_c_ui, Wl_o_ui], axis=1)
    Wr_ui_s = jnp.concatenate([Wr_i_ui, Wr_c_ui, Wr_o_ui], axis=1)
    Wl_iu_s = jnp.concatenate([Wl_i_iu, Wl_c_iu, Wl_o_iu], axis=1)
    Wr_iu_s = jnp.concatenate([Wr_i_iu, Wr_c_iu, Wr_o_iu], axis=1)
    bias_u = jnp.concatenate([b_i_user + bl_i_iu, b_c_user + bl_c_iu,
                              b_o_user + bl_o_iu]).reshape(1, DW)
    bias_i = jnp.concatenate([b_i_item + bl_i_ui, b_c_item + bl_c_ui,
                              b_o_item + bl_o_ui]).reshape(1, DW)

    h_u, h_i, c_u, c_i = _tc_dense(
        aui_a, aui_b, aiu_a, aiu_b, x_user, x_item,
        Wu, Wi, Wl_ui_s, Wr_ui_s, Wl_iu_s, Wr_iu_s, bias_u, bias_i)
    return (h_u, h_i, c_u, c_i)


# speedup vs baseline: 10.4385x; 1.5718x over previous
import functools

import jax
import jax.numpy as jnp
from jax import lax
from jax.experimental import pallas as pl
from jax.experimental.pallas import tpu as pltpu
from jax.experimental.pallas import tpu_sc as plsc

N = 10000
D = 128
E = 320000
DH = 64
W_ROW = 80
CHUNK = 128
TILES = 16
CHUNKS_PER_TILE = 160
E_PAD = TILES * CHUNKS_PER_TILE * CHUNK
N_PAD = N + 112
ROWS_PER_TILE = N_PAD // TILES
NBUF = 1
NG = 3
DW = NG * D


def _sc_segment_sums(src_ui, dst_ui, src_iu, dst_iu,
                     xu_a, xu_b, xi_a, xi_b, zrows):
    mesh = plsc.VectorSubcoreMesh(core_axis_name="c", subcore_axis_name="s")
    acc_sds = jax.ShapeDtypeStruct((N_PAD, W_ROW), jnp.float32)

    @functools.partial(
        pl.kernel,
        mesh=mesh,
        out_type=[acc_sds, acc_sds, acc_sds, acc_sds],
        scratch_types=[
            pltpu.VMEM((CHUNKS_PER_TILE, CHUNK), jnp.int32),
            pltpu.VMEM((CHUNKS_PER_TILE, CHUNK), jnp.int32),
            pltpu.VMEM_SHARED((N_PAD, W_ROW), jnp.float32),
        ] + [pltpu.VMEM((CHUNK, W_ROW), jnp.float32)] * NBUF
          + [pltpu.SemaphoreType.DMA] * NBUF,
        compiler_params=pltpu.CompilerParams(use_tc_tiling_on_sc=False),
    )
    def k(src_ui_h, dst_ui_h, src_iu_h, dst_iu_h,
          xu_a_h, xu_b_h, xi_a_h, xi_b_h, z_h,
          out_ui_a_h, out_ui_b_h, out_iu_a_h, out_iu_b_h,
          src_v, dst_v, acc_s, *bufs_and_sems):
        rows_bufs = bufs_and_sems[:NBUF]
        gsems = bufs_and_sems[NBUF:]
        core = lax.axis_index("c")
        sid = lax.axis_index("s")
        row_sl = pl.ds(sid * ROWS_PER_TILE, ROWS_PER_TILE)

        pltpu.sync_copy(z_h, acc_s.at[row_sl])

        def run(src_h, dst_h, tables, outs):
            idx_sl = pl.ds(sid * CHUNKS_PER_TILE, CHUNKS_PER_TILE)
            pltpu.sync_copy(src_h.at[idx_sl], src_v)
            pltpu.sync_copy(dst_h.at[idx_sl], dst_v)
            plsc.subcore_barrier()
            npp = CHUNKS_PER_TILE // 2
            for phase, (table_h, out_h) in enumerate(zip(tables, outs)):
                base = phase * npp

                def gdesc(j, b):
                    return pltpu.make_async_copy(
                        table_h.at[src_v.at[base + j]], rows_bufs[b],
                        gsems[b])

                for b in range(NBUF):
                    gdesc(b, b).start()

                def group(g, carry):
                    for b in range(NBUF):
                        j = g * NBUF + b
                        gdesc(j, b).wait()
                        pltpu.sync_copy(rows_bufs[b],
                                        acc_s.at[dst_v.at[base + j]],
                                        add=True)

                        @pl.when(j + NBUF < npp)
                        def _():
                            gdesc(j + NBUF, b).start()
                    return carry

                lax.fori_loop(0, npp // NBUF, group, 0)
                plsc.subcore_barrier()
                pltpu.sync_copy(acc_s.at[row_sl], out_h.at[row_sl])
                if phase == 0:
                    pltpu.sync_copy(z_h, acc_s.at[row_sl])
                    plsc.subcore_barrier()

        @pl.when(core == 0)
        def _():
            run(src_ui_h, dst_ui_h, (xu_a_h, xu_b_h),
                (out_ui_a_h, out_ui_b_h))

        @pl.when(core == 1)
        def _():
            run(src_iu_h, dst_iu_h, (xi_a_h, xi_b_h),
                (out_iu_a_h, out_iu_b_h))

    return k(src_ui, dst_ui, src_iu, dst_iu, xu_a, xu_b, xi_a, xi_b, zrows)


def _tc_body(aui_a_ref, aui_b_ref, aiu_a_ref, aiu_b_ref, xu_ref, xi_ref,
             wu_ref, wi_ref, wlui_ref, wrui_ref, wliu_ref, wriu_ref,
             bu_ref, bi_ref,
             hu_ref, hi_ref, cu_ref, ci_ref,
             Au, Bu, Ai, Bi):
    f32 = jnp.float32

    @pl.when(pl.program_id(0) == 0)
    def _():
        for g in range(NG):
            sl = slice(g * D, (g + 1) * D)
            Au[:, sl] = jnp.dot(wi_ref[:, sl], wliu_ref[:, sl],
                                preferred_element_type=f32)
            Bu[:, sl] = jnp.dot(wu_ref[:, sl], wriu_ref[:, sl],
                                preferred_element_type=f32)
            Ai[:, sl] = jnp.dot(wu_ref[:, sl], wlui_ref[:, sl],
                                preferred_element_type=f32)
            Bi[:, sl] = jnp.dot(wi_ref[:, sl], wrui_ref[:, sl],
                                preferred_element_type=f32)

    def halve(a_ref, b_ref, x_ref, A, B, bias_ref):
        inv = 1.0 / jnp.maximum(a_ref[:, DH:DH + 1], 1.0)
        agg_a = a_ref[:, 0:DH] * inv
        agg_b = b_ref[:, 0:DH] * inv
        P = (jnp.dot(agg_a, A[0:DH, :], preferred_element_type=f32)
             + jnp.dot(agg_b, A[DH:D, :], preferred_element_type=f32)
             + jnp.dot(x_ref[:], B[:], preferred_element_type=f32)
             + bias_ref[:])
        i_g = jax.nn.sigmoid(P[:, 0:D])
        t_g = jnp.tanh(P[:, D:2 * D])
        o_g = jax.nn.sigmoid(P[:, 2 * D:3 * D])
        c = i_g * t_g
        h = o_g * jnp.tanh(c)
        return h, c

    h_u, c_u = halve(aiu_a_ref, aiu_b_ref, xu_ref, Au, Bu, bu_ref)
    h_i, c_i = halve(aui_a_ref, aui_b_ref, xi_ref, Ai, Bi, bi_ref)
    hu_ref[:] = h_u
    cu_ref[:] = c_u
    hi_ref[:] = h_i
    ci_ref[:] = c_i


def _tc_dense(aui_a, aui_b, aiu_a, aiu_b, x_user, x_item,
              Wu, Wi, Wl_ui, Wr_ui, Wl_iu, Wr_iu, bias_u, bias_i):
    blk = 2000
    grid = (N // blk,)
    row_spec = pl.BlockSpec((blk, D), lambda k: (k, 0))
    acc_spec = pl.BlockSpec((blk, W_ROW), lambda k: (k, 0))
    w_spec = pl.BlockSpec((D, DW), lambda k: (0, 0))
    b_spec = pl.BlockSpec((1, DW), lambda k: (0, 0))
    out_sds = jax.ShapeDtypeStruct((N, D), jnp.float32)
    return pl.pallas_call(
        _tc_body,
        grid=grid,
        in_specs=[acc_spec, acc_spec, acc_spec, acc_spec, row_spec, row_spec,
                  w_spec, w_spec, w_spec, w_spec, w_spec, w_spec,
                  b_spec, b_spec],
        out_specs=[row_spec, row_spec, row_spec, row_spec],
        out_shape=[out_sds, out_sds, out_sds, out_sds],
        scratch_shapes=[pltpu.VMEM((D, DW), jnp.float32)] * 4,
    )(aui_a, aui_b, aiu_a, aiu_b, x_user, x_item,
      Wu, Wi, Wl_ui, Wr_ui, Wl_iu, Wr_iu, bias_u, bias_i)


def kernel(x_user, x_item, ei_ui, ei_iu, W_i_user, b_i_user, W_i_item, b_i_item, Wl_i_ui, bl_i_ui, Wr_i_ui, Wl_i_iu, bl_i_iu, Wr_i_iu, W_f_user, b_f_user, W_f_item, b_f_item, Wl_f_ui, bl_f_ui, Wr_f_ui, Wl_f_iu, bl_f_iu, Wr_f_iu, W_o_user, b_o_user, W_o_item, b_o_item, Wl_o_ui, bl_o_ui, Wr_o_ui, Wl_o_iu, bl_o_iu, Wr_o_iu, W_c_user, b_c_user, W_c_item, b_c_item, Wl_c_ui, bl_c_ui, Wr_c_ui, Wl_c_iu, bl_c_iu, Wr_c_iu):
    f32 = jnp.float32

    ones_col = jnp.ones((N, 1), f32)
    pad_cols = jnp.zeros((N, W_ROW - DH - 1), f32)
    xu_a = jnp.concatenate([x_user[:, :DH], ones_col, pad_cols], axis=1)
    xu_b = jnp.concatenate([x_user[:, DH:], ones_col, pad_cols], axis=1)
    xi_a = jnp.concatenate([x_item[:, :DH], ones_col, pad_cols], axis=1)
    xi_b = jnp.concatenate([x_item[:, DH:], ones_col, pad_cols], axis=1)

    pad_e = E_PAD - E
    src_pad = jnp.zeros((pad_e,), jnp.int32)
    dst_pad = jnp.full((pad_e,), N, jnp.int32)
    rows2d = (TILES * CHUNKS_PER_TILE, CHUNK)
    src_ui = jnp.concatenate([ei_ui[0], src_pad]).reshape(rows2d)
    dst_ui = jnp.concatenate([ei_ui[1], dst_pad]).reshape(rows2d)
    src_iu = jnp.concatenate([ei_iu[0], src_pad]).reshape(rows2d)
    dst_iu = jnp.concatenate([ei_iu[1], dst_pad]).reshape(rows2d)

    zrows = jnp.zeros((ROWS_PER_TILE, W_ROW), f32)

    aui_a, aui_b, aiu_a, aiu_b = _sc_segment_sums(
        src_ui, dst_ui, src_iu, dst_iu, xu_a, xu_b, xi_a, xi_b, zrows)

    Wu = jnp.concatenate([W_i_user, W_c_user, W_o_user], axis=1)
    Wi = jnp.concatenate([W_i_item, W_c_item, W_o_item], axis=1)
    Wl_ui_s = jnp.concatenate([Wl_i_ui, Wl_c_ui, Wl_o_ui], axis=1)
    Wr_ui_s = jnp.concatenate([Wr_i_ui, Wr_c_ui, Wr_o_ui], axis=1)
    Wl_iu_s = jnp.concatenate([Wl_i_iu, Wl_c_iu, Wl_o_iu], axis=1)
    Wr_iu_s = jnp.concatenate([Wr_i_iu, Wr_c_iu, Wr_o_iu], axis=1)
    bias_u = jnp.concatenate([b_i_user + bl_i_iu, b_c_user + bl_c_iu,
                              b_o_user + bl_o_iu]).reshape(1, DW)
    bias_i = jnp.concatenate([b_i_item + bl_i_ui, b_c_item + bl_c_ui,
                              b_o_item + bl_o_ui]).reshape(1, DW)

    h_u, h_i, c_u, c_i = _tc_dense(
        aui_a, aui_b, aiu_a, aiu_b, x_user, x_item,
        Wu, Wi, Wl_ui_s, Wr_ui_s, Wl_iu_s, Wr_iu_s, bias_u, bias_i)
    return (h_u, h_i, c_u, c_i)
